# baseline (speedup 1.0000x reference)
import functools

import jax
import jax.numpy as jnp
from jax import lax
from jax.experimental import pallas as pl
from jax.experimental.pallas import tpu as pltpu
from jax.experimental.pallas import tpu_sc as plsc

T = 2048
HIDDEN = 768
NUM_EXPERTS = 8
INTER = 256
GROUP = 4
BLK = 128
NB = T // BLK + NUM_EXPERTS
ROWS = NB * BLK
TCHUNK = 256

_NC = 2
_NS = 16
_NW = _NC * _NS
TPW = T // _NW


def _dispatch_body(x_ref, gate_w_ref, bias_ref, dst_ref, w_ref, bexp_ref):
    x = x_ref[...]
    logits = lax.dot_general(x, gate_w_ref[...], (((1,), (1,)), ((), ())),
                             preferred_element_type=jnp.float32)
    scores = jax.nn.sigmoid(logits)
    scores_c = scores + bias_ref[...]

    def top2sum(s4):
        a, b, c, d = (s4[:, 0], s4[:, 1], s4[:, 2], s4[:, 3])
        return jnp.maximum(
            jnp.maximum(jnp.maximum(a + b, a + c), jnp.maximum(a + d, b + c)),
            jnp.maximum(b + d, c + d))

    g0 = top2sum(scores_c[:, 0:GROUP])
    g1 = top2sum(scores_c[:, GROUP:2 * GROUP])
    sel0 = (g0 >= g1).astype(jnp.float32)[:, None]
    lane = lax.broadcasted_iota(jnp.int32, (T, NUM_EXPERTS), 1)
    in_g0 = (lane < GROUP).astype(jnp.float32)
    maskf = sel0 * in_g0 + (1.0 - sel0) * (1.0 - in_g0)
    masked = scores_c * maskf - 1e9 * (1.0 - maskf)

    m = jnp.max(masked, axis=1, keepdims=True)
    eq = (masked == m).astype(jnp.float32)
    tri = (lax.broadcasted_iota(jnp.int32, (NUM_EXPERTS, NUM_EXPERTS), 0)
           < lax.broadcasted_iota(jnp.int32, (NUM_EXPERTS, NUM_EXPERTS), 1)
           ).astype(jnp.float32)
    prior = lax.dot_general(eq, tri, (((1,), (0,)), ((), ())),
                            preferred_element_type=jnp.float32)
    onehot = eq * (prior == 0.0).astype(jnp.float32)

    w = jnp.sum(onehot * scores, axis=1, keepdims=True)
    w_ref[...] = w / (w + 1e-20)

    counts = jnp.sum(onehot, axis=0, keepdims=True)
    blocks = jnp.floor((counts + (BLK - 1)) * (1.0 / BLK))
    srow = BLK * lax.dot_general(blocks, tri, (((1,), (0,)), ((), ())),
                                 preferred_element_type=jnp.float32)
    ends_blk = lax.dot_general(
        blocks, (lax.broadcasted_iota(jnp.int32, (NUM_EXPERTS, NUM_EXPERTS), 0)
                 <= lax.broadcasted_iota(jnp.int32, (NUM_EXPERTS, NUM_EXPERTS), 1)
                 ).astype(jnp.float32),
        (((1,), (0,)), ((), ())), preferred_element_type=jnp.float32)

    rc = lax.broadcasted_iota(jnp.int32, (TCHUNK, TCHUNK), 0)
    cc = lax.broadcasted_iota(jnp.int32, (TCHUNK, TCHUNK), 1)
    tril = (rc > cc).astype(jnp.float32)
    carry = jnp.zeros((1, NUM_EXPERTS), dtype=jnp.float32)
    for i in range(T // TCHUNK):
        oc = onehot[i * TCHUNK:(i + 1) * TCHUNK, :]
        rank = lax.dot_general(tril, oc, (((1,), (0,)), ((), ())),
                               preferred_element_type=jnp.float32) + carry
        dstc = jnp.sum(oc * (srow + rank), axis=1, keepdims=True)
        dst_ref[i * TCHUNK:(i + 1) * TCHUNK, :] = dstc.astype(jnp.int32)
        carry = carry + jnp.sum(oc, axis=0, keepdims=True)

    biota = lax.broadcasted_iota(jnp.int32, (BLK, NUM_EXPERTS), 0).astype(jnp.float32)
    ind = (biota >= ends_blk).astype(jnp.float32)
    bexp = jnp.minimum(jnp.sum(ind, axis=1, keepdims=True),
                       float(NUM_EXPERTS - 1))
    bexp_ref[...] = bexp.astype(jnp.int32)


def _dispatch(x, gate_w, bias_row):
    return pl.pallas_call(
        _dispatch_body,
        in_specs=[
            pl.BlockSpec((T, HIDDEN), lambda: (0, 0)),
            pl.BlockSpec((NUM_EXPERTS, HIDDEN), lambda: (0, 0)),
            pl.BlockSpec((1, NUM_EXPERTS), lambda: (0, 0)),
        ],
        out_specs=[
            pl.BlockSpec((T, 1), lambda: (0, 0)),
            pl.BlockSpec((T, 1), lambda: (0, 0)),
            pl.BlockSpec((BLK, 1), lambda: (0, 0)),
        ],
        out_shape=[
            jax.ShapeDtypeStruct((T, 1), jnp.int32),
            jax.ShapeDtypeStruct((T, 1), jnp.float32),
            jax.ShapeDtypeStruct((BLK, 1), jnp.int32),
        ],
    )(x, gate_w, bias_row)


def _sc_scatter_body(x_hbm, dst_hbm, xs_hbm, idx_v, rows_v, sem):
    wid = lax.axis_index("s") * _NC + lax.axis_index("c")
    base = wid * TPW
    pltpu.sync_copy(dst_hbm.at[pl.ds(base, TPW)], idx_v)
    pltpu.sync_copy(x_hbm.at[pl.ds(base, TPW)], rows_v)
    pltpu.async_copy(rows_v, xs_hbm.at[idx_v], sem).wait()


def _sc_gather_body(y_hbm, dst_hbm, rp_hbm, idx_v, rows_v, sem):
    wid = lax.axis_index("s") * _NC + lax.axis_index("c")
    base = wid * TPW
    pltpu.sync_copy(dst_hbm.at[pl.ds(base, TPW)], idx_v)
    pltpu.async_copy(y_hbm.at[idx_v], rows_v, sem).wait()
    pltpu.sync_copy(rows_v, rp_hbm.at[pl.ds(base, TPW)])


@functools.lru_cache(maxsize=None)
def _sc_kernels():
    mesh = plsc.VectorSubcoreMesh(core_axis_name="c", subcore_axis_name="s",
                                  num_cores=_NC, num_subcores=_NS)
    scratch = [
        pltpu.VMEM((TPW,), jnp.int32),
        pltpu.VMEM((TPW, HIDDEN), jnp.float32),
        pltpu.SemaphoreType.DMA,
    ]
    scatter = pl.kernel(
        _sc_scatter_body,
        out_type=jax.ShapeDtypeStruct((ROWS, HIDDEN), jnp.float32),
        mesh=mesh, scratch_types=scratch)
    gather = pl.kernel(
        _sc_gather_body,
        out_type=jax.ShapeDtypeStruct((T, HIDDEN), jnp.float32),
        mesh=mesh, scratch_types=scratch)
    return scatter, gather


def _sc_scatter(x, dst):
    return _sc_kernels()[0](x, dst)


def _sc_gather(y, dst):
    return _sc_kernels()[1](y, dst)


def _ffn_body(bexp_ref, xs_ref, wgu_ref, wd_ref, y_ref):
    del bexp_ref
    xb = xs_ref[...]
    gu = lax.dot_general(xb, wgu_ref[0], (((1,), (0,)), ((), ())),
                         preferred_element_type=jnp.float32)
    g = gu[:, :INTER]
    u = gu[:, INTER:]
    h = g * jax.nn.sigmoid(g) * u
    y_ref[...] = lax.dot_general(h, wd_ref[0], (((1,), (0,)), ((), ())),
                                 preferred_element_type=jnp.float32)


def _grouped_ffn(bexp, xs, w_gate_up, w_down):
    grid_spec = pltpu.PrefetchScalarGridSpec(
        num_scalar_prefetch=1,
        grid=(NB,),
        in_specs=[
            pl.BlockSpec((BLK, HIDDEN), lambda b, be: (b, 0)),
            pl.BlockSpec((1, HIDDEN, 2 * INTER), lambda b, be: (be[b], 0, 0)),
            pl.BlockSpec((1, INTER, HIDDEN), lambda b, be: (be[b], 0, 0)),
        ],
        out_specs=pl.BlockSpec((BLK, HIDDEN), lambda b, be: (b, 0)),
    )
    return pl.pallas_call(
        _ffn_body,
        grid_spec=grid_spec,
        out_shape=jax.ShapeDtypeStruct((ROWS, HIDDEN), jnp.float32),
    )(bexp, xs, w_gate_up, w_down)


def _shared_body(x_ref, sgu_ref, sd_ref, out_ref):
    xb = x_ref[...]
    sgu = lax.dot_general(xb, sgu_ref[...], (((1,), (0,)), ((), ())),
                          preferred_element_type=jnp.float32)
    sg = sgu[:, :INTER]
    su = sgu[:, INTER:]
    sh = sg * jax.nn.sigmoid(sg) * su
    out_ref[...] = lax.dot_general(sh, sd_ref[...], (((1,), (0,)), ((), ())),
                                   preferred_element_type=jnp.float32)


def _shared_ffn(x, shared_gate_up, shared_down):
    tb = 256
    return pl.pallas_call(
        _shared_body,
        grid=(T // tb,),
        in_specs=[
            pl.BlockSpec((tb, HIDDEN), lambda i: (i, 0)),
            pl.BlockSpec((HIDDEN, 2 * INTER), lambda i: (0, 0)),
            pl.BlockSpec((INTER, HIDDEN), lambda i: (0, 0)),
        ],
        out_specs=pl.BlockSpec((tb, HIDDEN), lambda i: (i, 0)),
        out_shape=jax.ShapeDtypeStruct((T, HIDDEN), jnp.float32),
    )(x, shared_gate_up, shared_down)


def _combine_body(sh_ref, rp_ref, w_ref, out_ref):
    out_ref[...] = sh_ref[...] + w_ref[...] * rp_ref[...]


def _combine(shared, rp, wcol):
    tb = 512
    return pl.pallas_call(
        _combine_body,
        grid=(T // tb,),
        in_specs=[
            pl.BlockSpec((tb, HIDDEN), lambda i: (i, 0)),
            pl.BlockSpec((tb, HIDDEN), lambda i: (i, 0)),
            pl.BlockSpec((tb, 1), lambda i: (i, 0)),
        ],
        out_specs=pl.BlockSpec((tb, HIDDEN), lambda i: (i, 0)),
        out_shape=jax.ShapeDtypeStruct((T, HIDDEN), jnp.float32),
    )(shared, rp, wcol)


def kernel(hidden_states, gate_w, correction_bias, w_gate_up, w_down,
           shared_gate_up, shared_down):
    bias_row = correction_bias.reshape(1, NUM_EXPERTS)
    dst2, wcol, bexp_col = _dispatch(hidden_states, gate_w, bias_row)
    dst = dst2.reshape(T)
    bexp = bexp_col.reshape(BLK)[:NB]
    xs = _sc_scatter(hidden_states, dst)
    y = _grouped_ffn(bexp, xs, w_gate_up, w_down)
    shared = _shared_ffn(hidden_states, shared_gate_up, shared_down)
    rp = _sc_gather(y, dst)
    return _combine(shared, rp, wcol)


# speedup vs baseline: 1.1882x; 1.1882x over previous
import jax
import jax.numpy as jnp
from jax import lax
from jax.experimental import pallas as pl

T = 2048
HIDDEN = 768
NUM_EXPERTS = 8
INTER = 256
GROUP = 4
TBLK = 256


def _router_combine(xb, gate_w, bias_row):
    logits = lax.dot_general(xb, gate_w, (((1,), (1,)), ((), ())),
                             preferred_element_type=jnp.float32)
    scores = jax.nn.sigmoid(logits)
    scores_c = scores + bias_row

    def top2sum(s4):
        a, b, c, d = (s4[:, 0], s4[:, 1], s4[:, 2], s4[:, 3])
        return jnp.maximum(
            jnp.maximum(jnp.maximum(a + b, a + c), jnp.maximum(a + d, b + c)),
            jnp.maximum(b + d, c + d))

    g0 = top2sum(scores_c[:, 0:GROUP])
    g1 = top2sum(scores_c[:, GROUP:2 * GROUP])
    sel0 = (g0 >= g1).astype(jnp.float32)[:, None]
    lane = lax.broadcasted_iota(jnp.int32, (TBLK, NUM_EXPERTS), 1)
    in_g0 = (lane < GROUP).astype(jnp.float32)
    maskf = sel0 * in_g0 + (1.0 - sel0) * (1.0 - in_g0)
    masked = scores_c * maskf - 1e9 * (1.0 - maskf)

    m = jnp.max(masked, axis=1, keepdims=True)
    eq = (masked == m).astype(jnp.float32)
    tri = (lax.broadcasted_iota(jnp.int32, (NUM_EXPERTS, NUM_EXPERTS), 0)
           < lax.broadcasted_iota(jnp.int32, (NUM_EXPERTS, NUM_EXPERTS), 1)
           ).astype(jnp.float32)
    prior = lax.dot_general(eq, tri, (((1,), (0,)), ((), ())),
                            preferred_element_type=jnp.float32)
    onehot = eq * (prior == 0.0).astype(jnp.float32)

    w = jnp.sum(onehot * scores, axis=1, keepdims=True)
    w = w / (w + 1e-20)
    return onehot * w


def _moe_body(x_ref, gate_w_ref, bias_ref, wgu_ref, wd_ref, sgu_ref, sd_ref,
              out_ref):
    xb = x_ref[...]
    combine = _router_combine(xb, gate_w_ref[...], bias_ref[...])

    xb16 = xb.astype(jnp.bfloat16)
    acc = jnp.zeros((TBLK, HIDDEN), dtype=jnp.float32)
    for e in range(NUM_EXPERTS):
        gu = lax.dot_general(xb16, wgu_ref[e], (((1,), (0,)), ((), ())),
                             preferred_element_type=jnp.float32)
        g = gu[:, :INTER]
        u = gu[:, INTER:]
        h = (g * jax.nn.sigmoid(g) * u).astype(jnp.bfloat16)
        eo = lax.dot_general(h, wd_ref[e], (((1,), (0,)), ((), ())),
                             preferred_element_type=jnp.float32)
        acc = acc + combine[:, e][:, None] * eo

    sgu = lax.dot_general(xb16, sgu_ref[...], (((1,), (0,)), ((), ())),
                          preferred_element_type=jnp.float32)
    sg = sgu[:, :INTER]
    su = sgu[:, INTER:]
    sh = (sg * jax.nn.sigmoid(sg) * su).astype(jnp.bfloat16)
    shared = lax.dot_general(sh, sd_ref[...], (((1,), (0,)), ((), ())),
                             preferred_element_type=jnp.float32)
    out_ref[...] = acc + shared


def kernel(hidden_states, gate_w, correction_bias, w_gate_up, w_down,
           shared_gate_up, shared_down):
    bias_row = correction_bias.reshape(1, NUM_EXPERTS)
    wgu16 = w_gate_up.astype(jnp.bfloat16)
    wd16 = w_down.astype(jnp.bfloat16)
    sgu16 = shared_gate_up.astype(jnp.bfloat16)
    sd16 = shared_down.astype(jnp.bfloat16)
    grid = (T // TBLK,)
    return pl.pallas_call(
        _moe_body,
        grid=grid,
        in_specs=[
            pl.BlockSpec((TBLK, HIDDEN), lambda i: (i, 0)),
            pl.BlockSpec((NUM_EXPERTS, HIDDEN), lambda i: (0, 0)),
            pl.BlockSpec((1, NUM_EXPERTS), lambda i: (0, 0)),
            pl.BlockSpec((NUM_EXPERTS, HIDDEN, 2 * INTER), lambda i: (0, 0, 0)),
            pl.BlockSpec((NUM_EXPERTS, INTER, HIDDEN), lambda i: (0, 0, 0)),
            pl.BlockSpec((HIDDEN, 2 * INTER), lambda i: (0, 0)),
            pl.BlockSpec((INTER, HIDDEN), lambda i: (0, 0)),
        ],
        out_specs=pl.BlockSpec((TBLK, HIDDEN), lambda i: (i, 0)),
        out_shape=jax.ShapeDtypeStruct((T, HIDDEN), jnp.float32),
    )(hidden_states, gate_w, bias_row, wgu16, wd16, sgu16, sd16)


# speedup vs baseline: 1.7136x; 1.4421x over previous
import jax
import jax.numpy as jnp
from jax import lax
from jax.experimental import pallas as pl

T = 2048
HIDDEN = 768
NUM_EXPERTS = 8
INTER = 256
GROUP = 4
TBLK = 256


def _router_combine(xb, gate_w, bias_row):
    logits = lax.dot_general(xb, gate_w, (((1,), (1,)), ((), ())),
                             preferred_element_type=jnp.float32)
    scores = jax.nn.sigmoid(logits)
    scores_c = scores + bias_row

    def top2sum(s4):
        a, b, c, d = (s4[:, 0], s4[:, 1], s4[:, 2], s4[:, 3])
        return jnp.maximum(
            jnp.maximum(jnp.maximum(a + b, a + c), jnp.maximum(a + d, b + c)),
            jnp.maximum(b + d, c + d))

    g0 = top2sum(scores_c[:, 0:GROUP])
    g1 = top2sum(scores_c[:, GROUP:2 * GROUP])
    sel0 = (g0 >= g1).astype(jnp.float32)[:, None]
    lane = lax.broadcasted_iota(jnp.int32, (TBLK, NUM_EXPERTS), 1)
    in_g0 = (lane < GROUP).astype(jnp.float32)
    maskf = sel0 * in_g0 + (1.0 - sel0) * (1.0 - in_g0)
    masked = scores_c * maskf - 1e9 * (1.0 - maskf)

    m = jnp.max(masked, axis=1, keepdims=True)
    eq = (masked == m).astype(jnp.float32)
    tri = (lax.broadcasted_iota(jnp.int32, (NUM_EXPERTS, NUM_EXPERTS), 0)
           < lax.broadcasted_iota(jnp.int32, (NUM_EXPERTS, NUM_EXPERTS), 1)
           ).astype(jnp.float32)
    prior = lax.dot_general(eq, tri, (((1,), (0,)), ((), ())),
                            preferred_element_type=jnp.float32)
    onehot = eq * (prior == 0.0).astype(jnp.float32)

    w = jnp.sum(onehot * scores, axis=1, keepdims=True)
    w = w / (w + 1e-20)
    return onehot * w


def _moe_body(x_ref, gate_w_ref, bias_ref, wgu_ref, wd_ref, sgu_ref, sd_ref,
              out_ref):
    xb = x_ref[...]
    combine = _router_combine(xb, gate_w_ref[...], bias_ref[...])

    hs = []
    for e in range(NUM_EXPERTS):
        gu = lax.dot_general(xb, wgu_ref[e], (((1,), (0,)), ((), ())),
                             preferred_element_type=jnp.float32)
        g = gu[:, :INTER]
        u = gu[:, INTER:]
        h = g * jax.nn.sigmoid(g) * u
        hs.append(h * combine[:, e][:, None])
    h_all = jnp.concatenate(hs, axis=1)
    acc = lax.dot_general(h_all, wd_ref[...], (((1,), (0,)), ((), ())),
                          preferred_element_type=jnp.float32)

    sgu = lax.dot_general(xb, sgu_ref[...], (((1,), (0,)), ((), ())),
                          preferred_element_type=jnp.float32)
    sg = sgu[:, :INTER]
    su = sgu[:, INTER:]
    sh = sg * jax.nn.sigmoid(sg) * su
    shared = lax.dot_general(sh, sd_ref[...], (((1,), (0,)), ((), ())),
                             preferred_element_type=jnp.float32)
    out_ref[...] = acc + shared


def kernel(hidden_states, gate_w, correction_bias, w_gate_up, w_down,
           shared_gate_up, shared_down):
    bias_row = correction_bias.reshape(1, NUM_EXPERTS)
    wd_flat = w_down.reshape(NUM_EXPERTS * INTER, HIDDEN)
    grid = (T // TBLK,)
    return pl.pallas_call(
        _moe_body,
        grid=grid,
        in_specs=[
            pl.BlockSpec((TBLK, HIDDEN), lambda i: (i, 0)),
            pl.BlockSpec((NUM_EXPERTS, HIDDEN), lambda i: (0, 0)),
            pl.BlockSpec((1, NUM_EXPERTS), lambda i: (0, 0)),
            pl.BlockSpec((NUM_EXPERTS, HIDDEN, 2 * INTER), lambda i: (0, 0, 0)),
            pl.BlockSpec((NUM_EXPERTS * INTER, HIDDEN), lambda i: (0, 0)),
            pl.BlockSpec((HIDDEN, 2 * INTER), lambda i: (0, 0)),
            pl.BlockSpec((INTER, HIDDEN), lambda i: (0, 0)),
        ],
        out_specs=pl.BlockSpec((TBLK, HIDDEN), lambda i: (i, 0)),
        out_shape=jax.ShapeDtypeStruct((T, HIDDEN), jnp.float32),
    )(hidden_states, gate_w, bias_row, w_gate_up, wd_flat,
      shared_gate_up, shared_down)
